# Initial kernel scaffold; baseline (speedup 1.0000x reference)
import jax
import jax.numpy as jnp
from jax.experimental import pallas as pl


def kernel(gene_expression, gene_input_ids, bool_masked_pos, group_mtx, gene_embedding_table):
    raise NotImplementedError("write your pallas kernel here")


# speedup vs baseline: 2.9783x; 2.9783x over previous
import dataclasses
import functools

import jax
import jax.numpy as jnp
from jax import lax
from jax.experimental import pallas as pl
from jax.experimental.pallas import tpu as pltpu
from jax.experimental.pallas import tpu_sc as plsc

LANDMARK_GENES = 978
VOCAB_SIZE = 20000
HIDDEN_SIZE = 128
BATCH = 1024

N_ROWS = BATCH * LANDMARK_GENES
LANES = 16
CHUNK = 256


def _scale_rows(rows_vmem, gex_vmem, n_rows):
    @pl.loop(0, n_rows)
    def _(r):
        zeros = jnp.zeros((LANES,), jnp.int32)
        ridx = jnp.full((LANES,), r, jnp.int32)
        g = plsc.load_gather(gex_vmem, [zeros, ridx])
        for c in range(HIDDEN_SIZE // LANES):
            sl = (r, pl.ds(c * LANES, LANES))
            rows_vmem[sl] = rows_vmem[sl] * g


def _gex_embed(ids_flat, gex_flat, table):
    mesh = plsc.VectorSubcoreMesh(core_axis_name="c", subcore_axis_name="s")
    cp = pltpu.CompilerParams()
    if "needs_layout_passes" in pltpu.CompilerParams.__dataclass_fields__:
        cp = dataclasses.replace(cp, needs_layout_passes=False)

    @functools.partial(
        pl.kernel,
        out_type=jax.ShapeDtypeStruct((N_ROWS, HIDDEN_SIZE), jnp.float32),
        mesh=mesh,
        compiler_params=cp,
    )
    def k(table_hbm, ids_hbm, gex_hbm, out_hbm):
        def body(ids_vmem, gex_vmem, out_vmem):
            pltpu.sync_copy(table_hbm.at[ids_vmem.at[0]], out_vmem)
            _scale_rows(out_vmem, gex_vmem, CHUNK)

        pltpu.emit_pipeline(
            body,
            grid=(N_ROWS // CHUNK,),
            in_specs=[
                pl.BlockSpec((1, CHUNK), lambda i: (0, i)),
                pl.BlockSpec((1, CHUNK), lambda i: (0, i)),
            ],
            out_specs=[pl.BlockSpec((CHUNK, HIDDEN_SIZE), lambda i: (i, 0))],
            core_axis_name=("c", "s"),
            dimension_semantics=(pltpu.PARALLEL,),
        )(ids_hbm, gex_hbm, out_hbm)

    return k(table, ids_flat, gex_flat)


def kernel(gene_expression, gene_input_ids, bool_masked_pos, group_mtx, gene_embedding_table):
    del bool_masked_pos, group_mtx
    ids_flat = gene_input_ids.astype(jnp.int32).reshape(1, N_ROWS)
    gex_flat = gene_expression.astype(jnp.float32).reshape(1, N_ROWS)
    out = _gex_embed(ids_flat, gex_flat, gene_embedding_table)
    return out.reshape(BATCH, LANDMARK_GENES, HIDDEN_SIZE)


# speedup vs baseline: 6.3335x; 2.1266x over previous
import dataclasses
import functools

import jax
import jax.numpy as jnp
from jax import lax
from jax.experimental import pallas as pl
from jax.experimental.pallas import tpu as pltpu
from jax.experimental.pallas import tpu_sc as plsc

LANDMARK_GENES = 978
VOCAB_SIZE = 20000
HIDDEN_SIZE = 128
BATCH = 1024

LANES = 16
CHUNK = 256
N_CHUNKS = BATCH // CHUNK


def _scale_rows(rows_vmem, gex_vmem, n_rows):
    @pl.loop(0, n_rows)
    def _(r):
        zeros = jnp.zeros((LANES,), jnp.int32)
        ridx = jnp.full((LANES,), r, jnp.int32)
        g = plsc.load_gather(gex_vmem, [zeros, zeros, ridx])
        for c in range(HIDDEN_SIZE // LANES):
            sl = (r, pl.ds(c * LANES, LANES))
            rows_vmem[sl] = rows_vmem[sl] * g


def _gex_embed(ids_t, gex_t, table):
    mesh = plsc.VectorSubcoreMesh(core_axis_name="c", subcore_axis_name="s")
    cp = pltpu.CompilerParams()
    for _field, _val in (("needs_layout_passes", False),
                         ("use_tc_tiling_on_sc", False)):
        if _field in pltpu.CompilerParams.__dataclass_fields__:
            cp = dataclasses.replace(cp, **{_field: _val})

    @functools.partial(
        pl.kernel,
        out_type=jax.ShapeDtypeStruct(
            (LANDMARK_GENES, BATCH, HIDDEN_SIZE), jnp.float32
        ),
        mesh=mesh,
        compiler_params=cp,
    )
    def k(table_hbm, ids_hbm, gex_hbm, out_hbm):
        def body(ids_vmem, gex_vmem, out_vmem):
            pltpu.sync_copy(table_hbm.at[ids_vmem.at[0, 0]], out_vmem.at[0])
            _scale_rows(out_vmem.at[0], gex_vmem, CHUNK)

        pltpu.emit_pipeline(
            body,
            grid=(LANDMARK_GENES, N_CHUNKS),
            in_specs=[
                pl.BlockSpec((1, 1, CHUNK), lambda g, j: (g, 0, j)),
                pl.BlockSpec((1, 1, CHUNK), lambda g, j: (g, 0, j)),
            ],
            out_specs=[
                pl.BlockSpec((1, CHUNK, HIDDEN_SIZE), lambda g, j: (g, j, 0))
            ],
            core_axis_name=("c", "s"),
            dimension_semantics=(pltpu.PARALLEL, pltpu.PARALLEL),
        )(ids_hbm, gex_hbm, out_hbm)

    return k(table, ids_t, gex_t)


def kernel(gene_expression, gene_input_ids, bool_masked_pos, group_mtx, gene_embedding_table):
    del bool_masked_pos, group_mtx
    ids_t = gene_input_ids.astype(jnp.int32).T.reshape(
        LANDMARK_GENES, 1, BATCH)
    gex_t = gene_expression.astype(jnp.float32).T.reshape(
        LANDMARK_GENES, 1, BATCH)
    out_t = _gex_embed(ids_t, gex_t, gene_embedding_table)
    return jnp.transpose(out_t, (1, 0, 2))


# speedup vs baseline: 8.3266x; 1.3147x over previous
import dataclasses
import functools

import jax
import jax.numpy as jnp
from jax import lax
from jax.experimental import pallas as pl
from jax.experimental.pallas import tpu as pltpu
from jax.experimental.pallas import tpu_sc as plsc

LANDMARK_GENES = 978
VOCAB_SIZE = 20000
HIDDEN_SIZE = 128
BATCH = 1024

LANES = 16
CHUNK = 256
N_CHUNKS = BATCH // CHUNK


def _scale_rows(rows_vmem, gex_vmem, n_rows):
    @plsc.parallel_loop(0, n_rows, unroll=8)
    def _(r):
        zeros = jnp.zeros((LANES,), jnp.int32)
        ridx = jnp.full((LANES,), r, jnp.int32)
        g = plsc.load_gather(gex_vmem, [zeros, zeros, ridx])
        for c in range(HIDDEN_SIZE // LANES):
            sl = (r, pl.ds(c * LANES, LANES))
            rows_vmem[sl] = rows_vmem[sl] * g


def _gex_embed(ids_t, gex_t, table):
    mesh = plsc.VectorSubcoreMesh(core_axis_name="c", subcore_axis_name="s")
    cp = pltpu.CompilerParams()
    for _field, _val in (("needs_layout_passes", False),
                         ("use_tc_tiling_on_sc", False)):
        if _field in pltpu.CompilerParams.__dataclass_fields__:
            cp = dataclasses.replace(cp, **{_field: _val})

    @functools.partial(
        pl.kernel,
        out_type=jax.ShapeDtypeStruct(
            (LANDMARK_GENES, BATCH, HIDDEN_SIZE), jnp.float32
        ),
        mesh=mesh,
        compiler_params=cp,
    )
    def k(table_hbm, ids_hbm, gex_hbm, out_hbm):
        def body(ids_vmem, gex_vmem, out_vmem):
            pltpu.sync_copy(table_hbm.at[ids_vmem.at[0, 0]], out_vmem.at[0])
            _scale_rows(out_vmem.at[0], gex_vmem, CHUNK)

        pltpu.emit_pipeline(
            body,
            grid=(LANDMARK_GENES, N_CHUNKS),
            in_specs=[
                pl.BlockSpec((1, 1, CHUNK), lambda g, j: (g, 0, j)),
                pl.BlockSpec((1, 1, CHUNK), lambda g, j: (g, 0, j)),
            ],
            out_specs=[
                pl.BlockSpec((1, CHUNK, HIDDEN_SIZE), lambda g, j: (g, j, 0))
            ],
            core_axis_name=("c", "s"),
            dimension_semantics=(pltpu.PARALLEL, pltpu.PARALLEL),
        )(ids_hbm, gex_hbm, out_hbm)

    return k(table, ids_t, gex_t)


def kernel(gene_expression, gene_input_ids, bool_masked_pos, group_mtx, gene_embedding_table):
    del bool_masked_pos, group_mtx
    ids_t = gene_input_ids.astype(jnp.int32).T.reshape(
        LANDMARK_GENES, 1, BATCH)
    gex_t = gene_expression.astype(jnp.float32).T.reshape(
        LANDMARK_GENES, 1, BATCH)
    out_t = _gex_embed(ids_t, gex_t, gene_embedding_table)
    return jnp.transpose(out_t, (1, 0, 2))


# speedup vs baseline: 9.9578x; 1.1959x over previous
import dataclasses
import functools

import jax
import jax.numpy as jnp
from jax import lax
from jax.experimental import pallas as pl
from jax.experimental.pallas import tpu as pltpu
from jax.experimental.pallas import tpu_sc as plsc

LANDMARK_GENES = 978
VOCAB_SIZE = 20000
HIDDEN_SIZE = 128
BATCH = 1024

LANES = 16
CHUNK = 256
N_CHUNKS = BATCH // CHUNK


def _scale_rows(rows_vmem, gex_vmem, lo, n_rows):
    @plsc.parallel_loop(lo, lo + n_rows, unroll=8)
    def _(r):
        zeros = jnp.zeros((LANES,), jnp.int32)
        ridx = jnp.full((LANES,), r, jnp.int32)
        g = plsc.load_gather(gex_vmem, [zeros, zeros, ridx])
        for c in range(HIDDEN_SIZE // LANES):
            sl = (r, pl.ds(c * LANES, LANES))
            rows_vmem[sl] = rows_vmem[sl] * g


def _gex_embed(ids_t, gex_t, table):
    mesh = plsc.VectorSubcoreMesh(core_axis_name="c", subcore_axis_name="s")
    cp = pltpu.CompilerParams()
    for _field, _val in (("needs_layout_passes", False),
                         ("use_tc_tiling_on_sc", False)):
        if _field in pltpu.CompilerParams.__dataclass_fields__:
            cp = dataclasses.replace(cp, **{_field: _val})

    n_split = 4
    sub = CHUNK // n_split

    @functools.partial(
        pl.kernel,
        out_type=jax.ShapeDtypeStruct(
            (LANDMARK_GENES, BATCH, HIDDEN_SIZE), jnp.float32
        ),
        mesh=mesh,
        compiler_params=cp,
        scratch_types=[pltpu.SemaphoreType.DMA] * n_split,
    )
    def k(table_hbm, ids_hbm, gex_hbm, out_hbm, *sems):
        def body(ids_vmem, gex_vmem, out_vmem):
            copies = []
            for h in range(n_split):
                copies.append(pltpu.async_copy(
                    table_hbm.at[ids_vmem.at[0, 0, pl.ds(h * sub, sub)]],
                    out_vmem.at[0, pl.ds(h * sub, sub)],
                    sems[h],
                ))
            for h in range(n_split):
                copies[h].wait()
                _scale_rows(out_vmem.at[0], gex_vmem, h * sub, sub)

        pltpu.emit_pipeline(
            body,
            grid=(LANDMARK_GENES, N_CHUNKS),
            in_specs=[
                pl.BlockSpec((1, 1, CHUNK), lambda g, j: (g, 0, j)),
                pl.BlockSpec((1, 1, CHUNK), lambda g, j: (g, 0, j)),
            ],
            out_specs=[
                pl.BlockSpec((1, CHUNK, HIDDEN_SIZE), lambda g, j: (g, j, 0))
            ],
            core_axis_name=("c", "s"),
            dimension_semantics=(pltpu.PARALLEL, pltpu.PARALLEL),
        )(ids_hbm, gex_hbm, out_hbm)

    return k(table, ids_t, gex_t)


def kernel(gene_expression, gene_input_ids, bool_masked_pos, group_mtx, gene_embedding_table):
    del bool_masked_pos, group_mtx
    ids_t = gene_input_ids.astype(jnp.int32).T.reshape(
        LANDMARK_GENES, 1, BATCH)
    gex_t = gene_expression.astype(jnp.float32).T.reshape(
        LANDMARK_GENES, 1, BATCH)
    out_t = _gex_embed(ids_t, gex_t, gene_embedding_table)
    return jnp.transpose(out_t, (1, 0, 2))


# speedup vs baseline: 10.1575x; 1.0201x over previous
import dataclasses
import functools

import jax
import jax.numpy as jnp
from jax import lax
from jax.experimental import pallas as pl
from jax.experimental.pallas import tpu as pltpu
from jax.experimental.pallas import tpu_sc as plsc

LANDMARK_GENES = 978
VOCAB_SIZE = 20000
HIDDEN_SIZE = 128
BATCH = 1024

LANES = 16
CHUNK = 256
N_CHUNKS = BATCH // CHUNK


def _scale_rows(rows_vmem, gex_vmem, lo, n_rows):
    @plsc.parallel_loop(lo, lo + n_rows, unroll=16)
    def _(r):
        zeros = jnp.zeros((LANES,), jnp.int32)
        ridx = jnp.full((LANES,), r, jnp.int32)
        g = plsc.load_gather(gex_vmem, [zeros, zeros, ridx])
        for c in range(HIDDEN_SIZE // LANES):
            sl = (r, pl.ds(c * LANES, LANES))
            rows_vmem[sl] = rows_vmem[sl] * g


def _gex_embed(ids_t, gex_t, table):
    mesh = plsc.VectorSubcoreMesh(core_axis_name="c", subcore_axis_name="s")
    cp = pltpu.CompilerParams()
    for _field, _val in (("needs_layout_passes", False),
                         ("use_tc_tiling_on_sc", False)):
        if _field in pltpu.CompilerParams.__dataclass_fields__:
            cp = dataclasses.replace(cp, **{_field: _val})

    n_split = 8
    sub = CHUNK // n_split

    @functools.partial(
        pl.kernel,
        out_type=jax.ShapeDtypeStruct(
            (LANDMARK_GENES, BATCH, HIDDEN_SIZE), jnp.float32
        ),
        mesh=mesh,
        compiler_params=cp,
        scratch_types=[pltpu.SemaphoreType.DMA] * n_split,
    )
    def k(table_hbm, ids_hbm, gex_hbm, out_hbm, *sems):
        def body(ids_vmem, gex_vmem, out_vmem):
            copies = []
            for h in range(n_split):
                copies.append(pltpu.async_copy(
                    table_hbm.at[ids_vmem.at[0, 0, pl.ds(h * sub, sub)]],
                    out_vmem.at[0, pl.ds(h * sub, sub)],
                    sems[h],
                ))
            for h in range(n_split):
                copies[h].wait()
                _scale_rows(out_vmem.at[0], gex_vmem, h * sub, sub)

        pltpu.emit_pipeline(
            body,
            grid=(LANDMARK_GENES, N_CHUNKS),
            in_specs=[
                pl.BlockSpec((1, 1, CHUNK), lambda g, j: (g, 0, j)),
                pl.BlockSpec((1, 1, CHUNK), lambda g, j: (g, 0, j)),
            ],
            out_specs=[
                pl.BlockSpec((1, CHUNK, HIDDEN_SIZE), lambda g, j: (g, j, 0))
            ],
            core_axis_name=("c", "s"),
            dimension_semantics=(pltpu.PARALLEL, pltpu.PARALLEL),
        )(ids_hbm, gex_hbm, out_hbm)

    return k(table, ids_t, gex_t)


def kernel(gene_expression, gene_input_ids, bool_masked_pos, group_mtx, gene_embedding_table):
    del bool_masked_pos, group_mtx
    ids_t = gene_input_ids.astype(jnp.int32).T.reshape(
        LANDMARK_GENES, 1, BATCH)
    gex_t = gene_expression.astype(jnp.float32).T.reshape(
        LANDMARK_GENES, 1, BATCH)
    out_t = _gex_embed(ids_t, gex_t, gene_embedding_table)
    return jnp.transpose(out_t, (1, 0, 2))
